# baseline (speedup 1.0000x reference)
import functools

import jax
import jax.numpy as jnp
from jax import lax
from jax.experimental import pallas as pl
from jax.experimental.pallas import tpu as pltpu
from jax.experimental.pallas import tpu_sc as plsc

BATCH = 16384
IN_F = 512
OUT_F = 64

NC = 2
NS = 16
L = 16
NW = NC * NS
ROWS_W = BATCH // NW
CHUNK = 64
NCHUNK = ROWS_W // CHUNK
NG = OUT_F // L


def _sc_gather(x, columns):
    mesh = plsc.VectorSubcoreMesh(core_axis_name="c", subcore_axis_name="s")

    @functools.partial(
        pl.kernel,
        mesh=mesh,
        out_type=jax.ShapeDtypeStruct((BATCH, OUT_F), jnp.float32),
        compiler_params=pltpu.CompilerParams(
            needs_layout_passes=False, skip_device_barrier=True
        ),
        scratch_types=[
            pltpu.VMEM((OUT_F,), jnp.int32),
            pltpu.VMEM((CHUNK, IN_F), jnp.float32),
            pltpu.VMEM((CHUNK, IN_F), jnp.float32),
            pltpu.VMEM((CHUNK, OUT_F), jnp.float32),
            pltpu.VMEM((CHUNK, OUT_F), jnp.float32),
            pltpu.SemaphoreType.DMA,
            pltpu.SemaphoreType.DMA,
            pltpu.SemaphoreType.DMA,
            pltpu.SemaphoreType.DMA,
        ],
    )
    def k(x_hbm, cols_hbm, out_hbm, cols_v, in0, in1, ou0, ou1, is0, is1, os0, os1):
        wid = lax.axis_index("s") * NC + lax.axis_index("c")
        base = wid * ROWS_W
        pltpu.sync_copy(cols_hbm, cols_v)
        col_regs = [cols_v[pl.ds(g * L, L)] for g in range(NG)]
        ins = [in0, in1]
        outs = [ou0, ou1]
        isem = [is0, is1]
        osem = [os0, os1]

        def start_load(ci):
            row0 = base + ci * CHUNK
            return pltpu.async_copy(
                x_hbm.at[pl.ds(row0, CHUNK)], ins[ci % 2], isem[ci % 2]
            )

        loads = [None] * NCHUNK
        stores = [None] * NCHUNK
        loads[0] = start_load(0)
        for ci in range(NCHUNK):
            if ci + 1 < NCHUNK:
                loads[ci + 1] = start_load(ci + 1)
            loads[ci].wait()
            if ci >= 2:
                stores[ci - 2].wait()
            ib = ins[ci % 2]
            ob = outs[ci % 2]

            @plsc.parallel_loop(0, CHUNK, unroll=4)
            def row_body(r):
                ridx = jnp.zeros((L,), jnp.int32) + r
                for g in range(NG):
                    vals = plsc.load_gather(ib, [ridx, col_regs[g]])
                    ob[r, pl.ds(g * L, L)] = vals

            row0 = base + ci * CHUNK
            stores[ci] = pltpu.async_copy(
                ob, out_hbm.at[pl.ds(row0, CHUNK)], osem[ci % 2]
            )
        stores[NCHUNK - 2].wait()
        stores[NCHUNK - 1].wait()

    return k(x, columns)


def kernel(x, columns):
    return _sc_gather(x, columns)


# speedup vs baseline: 1.0009x; 1.0009x over previous
import functools

import jax
import jax.numpy as jnp
from jax import lax
from jax.experimental import pallas as pl
from jax.experimental.pallas import tpu as pltpu
from jax.experimental.pallas import tpu_sc as plsc

BATCH = 16384
IN_F = 512
OUT_F = 64

NC = 2
NS = 16
L = 16
NW = NC * NS
ROWS_W = BATCH // NW
CHUNK = 64
NCHUNK = ROWS_W // CHUNK
NG = OUT_F // L


def _sc_gather(x, columns):
    mesh = plsc.VectorSubcoreMesh(core_axis_name="c", subcore_axis_name="s")

    @functools.partial(
        pl.kernel,
        mesh=mesh,
        out_type=jax.ShapeDtypeStruct((BATCH, OUT_F), jnp.float32),
        compiler_params=pltpu.CompilerParams(
            needs_layout_passes=False,
            skip_device_barrier=True,
            use_tc_tiling_on_sc=True,
        ),
        scratch_types=[
            pltpu.VMEM((OUT_F,), jnp.int32),
            pltpu.VMEM((CHUNK, IN_F), jnp.float32),
            pltpu.VMEM((CHUNK, IN_F), jnp.float32),
            pltpu.VMEM((CHUNK, OUT_F), jnp.float32),
            pltpu.VMEM((CHUNK, OUT_F), jnp.float32),
            pltpu.SemaphoreType.DMA,
            pltpu.SemaphoreType.DMA,
            pltpu.SemaphoreType.DMA,
            pltpu.SemaphoreType.DMA,
        ],
    )
    def k(x_hbm, cols_hbm, out_hbm, cols_v, in0, in1, ou0, ou1, is0, is1, os0, os1):
        wid = lax.axis_index("s") * NC + lax.axis_index("c")
        base = wid * ROWS_W
        pltpu.sync_copy(cols_hbm, cols_v)
        col_regs = [cols_v[pl.ds(g * L, L)] for g in range(NG)]
        ins = [in0, in1]
        outs = [ou0, ou1]
        isem = [is0, is1]
        osem = [os0, os1]

        def start_load(ci):
            row0 = base + ci * CHUNK
            return pltpu.async_copy(
                x_hbm.at[pl.ds(row0, CHUNK)], ins[ci % 2], isem[ci % 2]
            )

        loads = [None] * NCHUNK
        stores = [None] * NCHUNK
        loads[0] = start_load(0)
        for ci in range(NCHUNK):
            if ci + 1 < NCHUNK:
                loads[ci + 1] = start_load(ci + 1)
            loads[ci].wait()
            if ci >= 2:
                stores[ci - 2].wait()
            ib = ins[ci % 2]
            ob = outs[ci % 2]

            @plsc.parallel_loop(0, CHUNK, unroll=4)
            def row_body(r):
                ridx = jnp.zeros((L,), jnp.int32) + r
                for g in range(NG):
                    vals = plsc.load_gather(ib, [ridx, col_regs[g]])
                    ob[r, pl.ds(g * L, L)] = vals

            row0 = base + ci * CHUNK
            stores[ci] = pltpu.async_copy(
                ob, out_hbm.at[pl.ds(row0, CHUNK)], osem[ci % 2]
            )
        stores[NCHUNK - 2].wait()
        stores[NCHUNK - 1].wait()

    return k(x, columns)


def kernel(x, columns):
    return _sc_gather(x, columns)


# speedup vs baseline: 1.0431x; 1.0421x over previous
import functools

import jax
import jax.numpy as jnp
from jax import lax
from jax.experimental import pallas as pl
from jax.experimental.pallas import tpu as pltpu
from jax.experimental.pallas import tpu_sc as plsc

BATCH = 16384
IN_F = 512
OUT_F = 64

NC = 2
NS = 16
L = 16
NW = NC * NS
ROWS_W = BATCH // NW
CHUNK = 64
NCHUNK = ROWS_W // CHUNK
OCHUNK = 128
NOC = ROWS_W // OCHUNK
NG = OUT_F // L


def _sc_gather(x, columns):
    mesh = plsc.VectorSubcoreMesh(core_axis_name="c", subcore_axis_name="s")

    @functools.partial(
        pl.kernel,
        mesh=mesh,
        out_type=jax.ShapeDtypeStruct((OUT_F, BATCH), jnp.float32),
        compiler_params=pltpu.CompilerParams(
            needs_layout_passes=False,
            skip_device_barrier=True,
        ),
        scratch_types=[
            pltpu.VMEM((OUT_F,), jnp.int32),
            pltpu.VMEM((CHUNK, IN_F), jnp.float32),
            pltpu.VMEM((CHUNK, IN_F), jnp.float32),
            pltpu.VMEM((OUT_F, OCHUNK), jnp.float32),
            pltpu.VMEM((OUT_F, OCHUNK), jnp.float32),
            pltpu.SemaphoreType.DMA,
            pltpu.SemaphoreType.DMA,
            pltpu.SemaphoreType.DMA,
            pltpu.SemaphoreType.DMA,
        ],
    )
    def k(x_hbm, cols_hbm, out_hbm, cols_v, in0, in1, ou0, ou1, is0, is1, os0, os1):
        wid = lax.axis_index("s") * NC + lax.axis_index("c")
        base = wid * ROWS_W
        pltpu.sync_copy(cols_hbm, cols_v)
        col_regs = [cols_v[pl.ds(g * L, L)] for g in range(NG)]
        out_cols = [lax.iota(jnp.int32, L) + g * L for g in range(NG)]
        ins = [in0, in1]
        outs = [ou0, ou1]
        isem = [is0, is1]
        osem = [os0, os1]

        def start_load(ci):
            row0 = base + ci * CHUNK
            return pltpu.async_copy(
                x_hbm.at[pl.ds(row0, CHUNK)], ins[ci % 2], isem[ci % 2]
            )

        loads = [None] * NCHUNK
        stores = [None] * NOC
        loads[0] = start_load(0)
        for oc in range(NOC):
            if oc >= 2:
                stores[oc - 2].wait()
            ob = outs[oc % 2]
            for h in range(2):
                ci = oc * 2 + h
                if ci + 1 < NCHUNK:
                    loads[ci + 1] = start_load(ci + 1)
                loads[ci].wait()
                ib = ins[ci % 2]

                @plsc.parallel_loop(0, CHUNK, unroll=4)
                def row_body(r):
                    ridx = jnp.zeros((L,), jnp.int32) + r
                    cidx = ridx + h * CHUNK
                    for g in range(NG):
                        vals = plsc.load_gather(ib, [ridx, col_regs[g]])
                        plsc.store_scatter(ob, [out_cols[g], cidx], vals)

            row0 = base + oc * OCHUNK
            stores[oc] = pltpu.async_copy(
                ob, out_hbm.at[:, pl.ds(row0, OCHUNK)], osem[oc % 2]
            )
        stores[NOC - 2].wait()
        stores[NOC - 1].wait()

    return k(x, columns)


def kernel(x, columns):
    return _sc_gather(x, columns).T
